# baseline (speedup 1.0000x reference)
import functools

import jax
import jax.numpy as jnp
from jax import lax
from jax.experimental import pallas as pl
from jax.experimental.pallas import tpu as pltpu
from jax.experimental.pallas import tpu_sc as plsc

B = 16384
E = 36
N = 12
D = 128
LANES = 16
NC, NS = 2, 16
NW = NC * NS
P = 4
BC = B // P
PER_W = BC // NW
CB = 8
G = PER_W // CB


def _sc_agg_body(p, x_hbm, w_hbm, o_hbm, wv, xv0, xv1, ov0, ov1,
                 si0, si1, so0, so1):
    wid = lax.axis_index("s") * NC + lax.axis_index("c")
    base = p * BC + wid * PER_W
    obase = wid * PER_W
    pltpu.sync_copy(w_hbm, wv)

    xvs, ovs, sis, sos = (xv0, xv1), (ov0, ov1), (si0, si1), (so0, so1)

    def in_copy(g, k):
        return pltpu.make_async_copy(
            x_hbm.at[:, pl.ds(base + g * CB, CB), :], xvs[k], sis[k])

    def out_copy(g, k):
        return pltpu.make_async_copy(
            ovs[k], o_hbm.at[:, pl.ds(obase + g * CB, CB), :], sos[k])

    def compute(k):
        xv, ov = xvs[k], ovs[k]

        def body(i, _):
            for c in range(D // LANES):
                sl = pl.ds(c * LANES, LANES)
                accs = [None] * N
                for e in range(E):
                    u, v = e // 6, 6 + e % 6
                    pr = xv[e, i, sl] * wv[pl.ds(e * LANES, LANES)]
                    accs[u] = pr if accs[u] is None else accs[u] + pr
                    accs[v] = pr if accs[v] is None else accs[v] + pr
                for n in range(N):
                    ov[n, i, sl] = accs[n]
            return 0

        lax.fori_loop(0, CB, body, 0)

    in_copy(0, 0).start()
    in_copy(1, 1).start()

    def step(s, _):
        for k in range(2):
            g = s * 2 + k
            in_copy(g, k).wait()

            @pl.when(g >= 2)
            def _():
                out_copy(g - 2, k).wait()

            compute(k)
            out_copy(g, k).start()

            @pl.when(g + 2 < G)
            def _():
                in_copy(g + 2, k).start()
        return 0

    lax.fori_loop(0, G // 2, step, 0)
    out_copy(G - 2, 0).wait()
    out_copy(G - 1, 1).wait()


def _sc_aggregate(p, x_t, w_bcast):
    mesh = plsc.VectorSubcoreMesh(
        core_axis_name="c", subcore_axis_name="s", num_cores=NC, num_subcores=NS)
    f = pl.kernel(
        functools.partial(_sc_agg_body, p),
        out_type=jax.ShapeDtypeStruct((N, BC, D), jnp.float32),
        mesh=mesh,
        scratch_types=[
            pltpu.VMEM((E * LANES,), jnp.float32),
            pltpu.VMEM((E, CB, D), jnp.float32),
            pltpu.VMEM((E, CB, D), jnp.float32),
            pltpu.VMEM((N, CB, D), jnp.float32),
            pltpu.VMEM((N, CB, D), jnp.float32),
            pltpu.SemaphoreType.DMA,
            pltpu.SemaphoreType.DMA,
            pltpu.SemaphoreType.DMA,
            pltpu.SemaphoreType.DMA,
        ],
        compiler_params=pltpu.CompilerParams(use_tc_tiling_on_sc=True),
    )
    return f(x_t, w_bcast)


BMR = 4096


def _mm_body_first(x_ref, wt_ref, b_ref, o_ref):
    acc = jnp.dot(x_ref[:], wt_ref[:], preferred_element_type=jnp.float32)
    o_ref[:] = jnp.maximum(acc + b_ref[:], 0.0)


def _mm_body_chain(prev_ref, x_ref, wt_ref, b_ref, o_ref):
    del prev_ref
    acc = jnp.dot(x_ref[:], wt_ref[:], preferred_element_type=jnp.float32)
    o_ref[:] = jnp.maximum(acc + b_ref[:], 0.0)


def _tc_linear_chunk(p, nf2d, wt, b2d, prev):
    jb = BC // BMR
    in_map = lambda n, j: (n * jb + j, 0)
    out_map = lambda n, j: (n * (B // BMR) + p * jb + j, 0)
    x_spec = pl.BlockSpec((BMR, D), in_map)
    w_spec = pl.BlockSpec((D, D), lambda n, j: (0, 0))
    b_spec = pl.BlockSpec((1, D), lambda n, j: (0, 0))
    o_spec = pl.BlockSpec((BMR, D), out_map)
    out_shape = jax.ShapeDtypeStruct((N * B, D), jnp.float32)
    params = pltpu.CompilerParams(dimension_semantics=("arbitrary", "arbitrary"))
    if prev is None:
        return pl.pallas_call(
            _mm_body_first,
            grid=(N, jb),
            in_specs=[x_spec, w_spec, b_spec],
            out_specs=o_spec,
            out_shape=out_shape,
            compiler_params=params,
        )(nf2d, wt, b2d)
    return pl.pallas_call(
        _mm_body_chain,
        grid=(N, jb),
        in_specs=[pl.BlockSpec(memory_space=pl.ANY), x_spec, w_spec, b_spec],
        out_specs=o_spec,
        out_shape=out_shape,
        input_output_aliases={0: 0},
        compiler_params=params,
    )(prev, nf2d, wt, b2d)


def kernel(edge_feats, edge_weights, W, b):
    w_sp = jax.nn.softplus(edge_weights.astype(jnp.float32))
    w_bcast = jnp.broadcast_to(w_sp[:, None], (E, LANES)).reshape(-1)
    x_t = jnp.transpose(edge_feats, (1, 0, 2))
    wt = W.T
    b2d = b.reshape(1, D)
    nfs = [_sc_aggregate(p, x_t, w_bcast) for p in range(P)]
    out2d = None
    for p in range(P):
        out2d = _tc_linear_chunk(p, nfs[p].reshape(N * BC, D), wt, b2d, out2d)
    return jnp.transpose(out2d.reshape(N, B, D), (1, 0, 2))


# speedup vs baseline: 1.3659x; 1.3659x over previous
import functools

import jax
import jax.numpy as jnp
from jax import lax
from jax.experimental import pallas as pl
from jax.experimental.pallas import tpu as pltpu
from jax.experimental.pallas import tpu_sc as plsc

B = 16384
E = 36
N = 12
D = 128
LANES = 16
NC, NS = 2, 16
NW = NC * NS

BSC = 8192
P = 4
BC = BSC // P
PER_W = BC // NW
CB = 8
G = PER_W // CB

BTC = B - BSC
BMB = 512
BMC = BC

_EDGES_OF_NODE = [[6 * n + j for j in range(6)] for n in range(6)] + \
                 [[6 * i + j for i in range(6)] for j in range(6)]


def _sc_agg_body(p, x_hbm, w_hbm, o_hbm, wv, xv0, xv1, ov0, ov1,
                 si0, si1, so0, so1):
    wid = lax.axis_index("s") * NC + lax.axis_index("c")
    base = p * BC + wid * PER_W
    obase = wid * PER_W
    pltpu.sync_copy(w_hbm, wv)

    xvs, ovs, sis, sos = (xv0, xv1), (ov0, ov1), (si0, si1), (so0, so1)

    def in_copy(g, k):
        return pltpu.make_async_copy(
            x_hbm.at[:, pl.ds(base + g * CB, CB), :], xvs[k], sis[k])

    def out_copy(g, k):
        return pltpu.make_async_copy(
            ovs[k], o_hbm.at[:, pl.ds(obase + g * CB, CB), :], sos[k])

    def compute(k):
        xv, ov = xvs[k], ovs[k]

        def body(i, _):
            for c in range(D // LANES):
                sl = pl.ds(c * LANES, LANES)
                accs = [None] * N
                for e in range(E):
                    u, v = e // 6, 6 + e % 6
                    pr = xv[e, i, sl] * wv[pl.ds(e * LANES, LANES)]
                    accs[u] = pr if accs[u] is None else accs[u] + pr
                    accs[v] = pr if accs[v] is None else accs[v] + pr
                for n in range(N):
                    ov[n, i, sl] = accs[n]
            return 0

        lax.fori_loop(0, CB, body, 0)

    in_copy(0, 0).start()
    in_copy(1, 1).start()

    def step(s, _):
        for k in range(2):
            g = s * 2 + k
            in_copy(g, k).wait()

            @pl.when(g >= 2)
            def _():
                out_copy(g - 2, k).wait()

            compute(k)
            out_copy(g, k).start()

            @pl.when(g + 2 < G)
            def _():
                in_copy(g + 2, k).start()
        return 0

    lax.fori_loop(0, G // 2, step, 0)
    out_copy(G - 2, 0).wait()
    out_copy(G - 1, 1).wait()


def _sc_aggregate(p, x_t, w_bcast):
    mesh = plsc.VectorSubcoreMesh(
        core_axis_name="c", subcore_axis_name="s", num_cores=NC, num_subcores=NS)
    f = pl.kernel(
        functools.partial(_sc_agg_body, p),
        out_type=jax.ShapeDtypeStruct((N, BC, D), jnp.float32),
        mesh=mesh,
        scratch_types=[
            pltpu.VMEM((E * LANES,), jnp.float32),
            pltpu.VMEM((E, CB, D), jnp.float32),
            pltpu.VMEM((E, CB, D), jnp.float32),
            pltpu.VMEM((N, CB, D), jnp.float32),
            pltpu.VMEM((N, CB, D), jnp.float32),
            pltpu.SemaphoreType.DMA,
            pltpu.SemaphoreType.DMA,
            pltpu.SemaphoreType.DMA,
            pltpu.SemaphoreType.DMA,
        ],
        compiler_params=pltpu.CompilerParams(use_tc_tiling_on_sc=True),
    )
    return f(x_t, w_bcast)


def _tc_fused_body(x_ref, w_ref, wt_ref, b_ref, o_ref):
    for n in range(N):
        acc = None
        for e in _EDGES_OF_NODE[n]:
            t = x_ref[e] * w_ref[0, e]
            acc = t if acc is None else acc + t
        r = jnp.dot(acc, wt_ref[:], preferred_element_type=jnp.float32)
        o_ref[n] = jnp.maximum(r + b_ref[:], 0.0)


def _tc_fused(x_t, w_sp2d, wt, b2d):
    nb = BTC // BMB
    return pl.pallas_call(
        _tc_fused_body,
        grid=(nb,),
        in_specs=[
            pl.BlockSpec((E, BMB, D), lambda j: (0, BSC // BMB + j, 0)),
            pl.BlockSpec((1, E), lambda j: (0, 0)),
            pl.BlockSpec((D, D), lambda j: (0, 0)),
            pl.BlockSpec((1, D), lambda j: (0, 0)),
        ],
        out_specs=pl.BlockSpec((N, BMB, D), lambda j: (0, BSC // BMB + j, 0)),
        out_shape=jax.ShapeDtypeStruct((N, B, D), jnp.float32),
        compiler_params=pltpu.CompilerParams(
            dimension_semantics=("arbitrary",)),
    )(x_t, w_sp2d, wt, b2d)


def _mm_chunk_body(prev_ref, x_ref, wt_ref, b_ref, o_ref):
    del prev_ref
    acc = jnp.dot(x_ref[0], wt_ref[:], preferred_element_type=jnp.float32)
    o_ref[0] = jnp.maximum(acc + b_ref[:], 0.0)


def _tc_chunk(p, nf, wt, b2d, prev):
    jb = BC // BMC
    return pl.pallas_call(
        _mm_chunk_body,
        grid=(N, jb),
        in_specs=[
            pl.BlockSpec(memory_space=pl.ANY),
            pl.BlockSpec((1, BMC, D), lambda n, j: (n, j, 0)),
            pl.BlockSpec((D, D), lambda n, j: (0, 0)),
            pl.BlockSpec((1, D), lambda n, j: (0, 0)),
        ],
        out_specs=pl.BlockSpec((1, BMC, D),
                               lambda n, j: (n, (p * BC) // BMC + j, 0)),
        out_shape=jax.ShapeDtypeStruct((N, B, D), jnp.float32),
        input_output_aliases={0: 0},
        compiler_params=pltpu.CompilerParams(
            dimension_semantics=("arbitrary", "arbitrary")),
    )(prev, nf, wt, b2d)


def kernel(edge_feats, edge_weights, W, b):
    w_sp = jax.nn.softplus(edge_weights.astype(jnp.float32))
    w_bcast = jnp.broadcast_to(w_sp[:, None], (E, LANES)).reshape(-1)
    w_sp2d = w_sp.reshape(1, E)
    x_t = jnp.transpose(edge_feats, (1, 0, 2))
    wt = W.T
    b2d = b.reshape(1, D)
    nfs = [_sc_aggregate(p, x_t, w_bcast) for p in range(P)]
    out3d = _tc_fused(x_t, w_sp2d, wt, b2d)
    for p in range(P):
        out3d = _tc_chunk(p, nfs[p], wt, b2d, out3d)
    return jnp.transpose(out3d, (1, 0, 2))


# speedup vs baseline: 1.5632x; 1.1444x over previous
import functools

import jax
import jax.numpy as jnp
from jax import lax
from jax.experimental import pallas as pl
from jax.experimental.pallas import tpu as pltpu
from jax.experimental.pallas import tpu_sc as plsc

B = 16384
E = 36
N = 12
D = 128
LANES = 16
NC, NS = 2, 16
NW = NC * NS

BSC = 6144
P = 2
BC = BSC // P
PER_W = BC // NW
CB = 8
G = PER_W // CB

BTC = B - BSC
BMB = 1024
BMC = BC

_EDGES_OF_NODE = [[6 * n + j for j in range(6)] for n in range(6)] + \
                 [[6 * i + j for i in range(6)] for j in range(6)]


def _sc_agg_body(p, x_hbm, w_hbm, o_hbm, wv, xv0, xv1, ov0, ov1,
                 si0, si1, so0, so1):
    wid = lax.axis_index("s") * NC + lax.axis_index("c")
    base = p * BC + wid * PER_W
    obase = wid * PER_W
    pltpu.sync_copy(w_hbm, wv)

    xvs, ovs, sis, sos = (xv0, xv1), (ov0, ov1), (si0, si1), (so0, so1)

    def in_copy(g, k):
        return pltpu.make_async_copy(
            x_hbm.at[:, pl.ds(base + g * CB, CB), :], xvs[k], sis[k])

    def out_copy(g, k):
        return pltpu.make_async_copy(
            ovs[k], o_hbm.at[:, pl.ds(obase + g * CB, CB), :], sos[k])

    def compute(k):
        xv, ov = xvs[k], ovs[k]

        def body(i, _):
            for c in range(D // LANES):
                sl = pl.ds(c * LANES, LANES)
                accs = [None] * N
                for e in range(E):
                    u, v = e // 6, 6 + e % 6
                    pr = xv[e, i, sl] * wv[pl.ds(e * LANES, LANES)]
                    accs[u] = pr if accs[u] is None else accs[u] + pr
                    accs[v] = pr if accs[v] is None else accs[v] + pr
                for n in range(N):
                    ov[n, i, sl] = accs[n]
            return 0

        lax.fori_loop(0, CB, body, 0)

    in_copy(0, 0).start()
    in_copy(1, 1).start()

    def step(s, _):
        for k in range(2):
            g = s * 2 + k
            in_copy(g, k).wait()

            @pl.when(g >= 2)
            def _():
                out_copy(g - 2, k).wait()

            compute(k)
            out_copy(g, k).start()

            @pl.when(g + 2 < G)
            def _():
                in_copy(g + 2, k).start()
        return 0

    lax.fori_loop(0, G // 2, step, 0)
    out_copy(G - 2, 0).wait()
    out_copy(G - 1, 1).wait()


def _sc_aggregate(p, x_t, w_bcast):
    mesh = plsc.VectorSubcoreMesh(
        core_axis_name="c", subcore_axis_name="s", num_cores=NC, num_subcores=NS)
    f = pl.kernel(
        functools.partial(_sc_agg_body, p),
        out_type=jax.ShapeDtypeStruct((N, BC, D), jnp.float32),
        mesh=mesh,
        scratch_types=[
            pltpu.VMEM((E * LANES,), jnp.float32),
            pltpu.VMEM((E, CB, D), jnp.float32),
            pltpu.VMEM((E, CB, D), jnp.float32),
            pltpu.VMEM((N, CB, D), jnp.float32),
            pltpu.VMEM((N, CB, D), jnp.float32),
            pltpu.SemaphoreType.DMA,
            pltpu.SemaphoreType.DMA,
            pltpu.SemaphoreType.DMA,
            pltpu.SemaphoreType.DMA,
        ],
        compiler_params=pltpu.CompilerParams(use_tc_tiling_on_sc=True),
    )
    return f(x_t, w_bcast)


def _tc_fused_body(x_ref, w_ref, wt_ref, b_ref, o_ref):
    for n in range(N):
        acc = None
        for e in _EDGES_OF_NODE[n]:
            t = x_ref[e] * w_ref[0, e]
            acc = t if acc is None else acc + t
        r = jnp.dot(acc, wt_ref[:], preferred_element_type=jnp.float32)
        o_ref[n] = jnp.maximum(r + b_ref[:], 0.0)


def _tc_fused(x_t, w_sp2d, wt, b2d):
    nb = BTC // BMB
    return pl.pallas_call(
        _tc_fused_body,
        grid=(nb,),
        in_specs=[
            pl.BlockSpec((E, BMB, D), lambda j: (0, BSC // BMB + j, 0)),
            pl.BlockSpec((1, E), lambda j: (0, 0)),
            pl.BlockSpec((D, D), lambda j: (0, 0)),
            pl.BlockSpec((1, D), lambda j: (0, 0)),
        ],
        out_specs=pl.BlockSpec((N, BMB, D), lambda j: (0, BSC // BMB + j, 0)),
        out_shape=jax.ShapeDtypeStruct((N, B, D), jnp.float32),
        compiler_params=pltpu.CompilerParams(
            dimension_semantics=("arbitrary",)),
    )(x_t, w_sp2d, wt, b2d)


def _mm_chunk_body(prev_ref, x_ref, wt_ref, b_ref, o_ref):
    del prev_ref
    acc = jnp.dot(x_ref[0], wt_ref[:], preferred_element_type=jnp.float32)
    o_ref[0] = jnp.maximum(acc + b_ref[:], 0.0)


def _tc_chunk(p, nf, wt, b2d, prev):
    jb = BC // BMC
    return pl.pallas_call(
        _mm_chunk_body,
        grid=(N, jb),
        in_specs=[
            pl.BlockSpec(memory_space=pl.ANY),
            pl.BlockSpec((1, BMC, D), lambda n, j: (n, j, 0)),
            pl.BlockSpec((D, D), lambda n, j: (0, 0)),
            pl.BlockSpec((1, D), lambda n, j: (0, 0)),
        ],
        out_specs=pl.BlockSpec((1, BMC, D),
                               lambda n, j: (n, (p * BC) // BMC + j, 0)),
        out_shape=jax.ShapeDtypeStruct((N, B, D), jnp.float32),
        input_output_aliases={0: 0},
        compiler_params=pltpu.CompilerParams(
            dimension_semantics=("arbitrary", "arbitrary")),
    )(prev, nf, wt, b2d)


def kernel(edge_feats, edge_weights, W, b):
    w_sp = jax.nn.softplus(edge_weights.astype(jnp.float32))
    w_bcast = jnp.broadcast_to(w_sp[:, None], (E, LANES)).reshape(-1)
    w_sp2d = w_sp.reshape(1, E)
    x_t = jnp.transpose(edge_feats, (1, 0, 2))
    wt = W.T
    b2d = b.reshape(1, D)
    nfs = [_sc_aggregate(p, x_t, w_bcast) for p in range(P)]
    out3d = _tc_fused(x_t, w_sp2d, wt, b2d)
    for p in range(P):
        out3d = _tc_chunk(p, nfs[p], wt, b2d, out3d)
    return jnp.transpose(out3d, (1, 0, 2))


# speedup vs baseline: 1.5940x; 1.0197x over previous
import functools

import jax
import jax.numpy as jnp
from jax import lax
from jax.experimental import pallas as pl
from jax.experimental.pallas import tpu as pltpu
from jax.experimental.pallas import tpu_sc as plsc

B = 16384
E = 36
N = 12
D = 128
LANES = 16
NC, NS = 2, 16
NW = NC * NS

BSC = 7168
P = 1
BC = BSC // P
PER_W = BC // NW
CB = 8
G = PER_W // CB

BTC = B - BSC
BMB = 1024
BMC = BC

_EDGES_OF_NODE = [[6 * n + j for j in range(6)] for n in range(6)] + \
                 [[6 * i + j for i in range(6)] for j in range(6)]


def _sc_agg_body(p, x_hbm, w_hbm, o_hbm, wv, xv0, xv1, ov0, ov1,
                 si0, si1, so0, so1):
    wid = lax.axis_index("s") * NC + lax.axis_index("c")
    base = p * BC + wid * PER_W
    obase = wid * PER_W
    pltpu.sync_copy(w_hbm, wv)

    xvs, ovs, sis, sos = (xv0, xv1), (ov0, ov1), (si0, si1), (so0, so1)

    def in_copy(g, k):
        return pltpu.make_async_copy(
            x_hbm.at[:, pl.ds(base + g * CB, CB), :], xvs[k], sis[k])

    def out_copy(g, k):
        return pltpu.make_async_copy(
            ovs[k], o_hbm.at[:, pl.ds(obase + g * CB, CB), :], sos[k])

    def compute(k):
        xv, ov = xvs[k], ovs[k]

        def body(i, _):
            for c in range(D // LANES):
                sl = pl.ds(c * LANES, LANES)
                accs = [None] * N
                for e in range(E):
                    u, v = e // 6, 6 + e % 6
                    pr = xv[e, i, sl] * wv[pl.ds(e * LANES, LANES)]
                    accs[u] = pr if accs[u] is None else accs[u] + pr
                    accs[v] = pr if accs[v] is None else accs[v] + pr
                for n in range(N):
                    ov[n, i, sl] = accs[n]
            return 0

        lax.fori_loop(0, CB, body, 0)

    in_copy(0, 0).start()
    in_copy(1, 1).start()

    def step(s, _):
        for k in range(2):
            g = s * 2 + k
            in_copy(g, k).wait()

            @pl.when(g >= 2)
            def _():
                out_copy(g - 2, k).wait()

            compute(k)
            out_copy(g, k).start()

            @pl.when(g + 2 < G)
            def _():
                in_copy(g + 2, k).start()
        return 0

    lax.fori_loop(0, G // 2, step, 0)
    out_copy(G - 2, 0).wait()
    out_copy(G - 1, 1).wait()


def _sc_aggregate(p, x_t, w_bcast):
    mesh = plsc.VectorSubcoreMesh(
        core_axis_name="c", subcore_axis_name="s", num_cores=NC, num_subcores=NS)
    f = pl.kernel(
        functools.partial(_sc_agg_body, p),
        out_type=jax.ShapeDtypeStruct((N, BC, D), jnp.float32),
        mesh=mesh,
        scratch_types=[
            pltpu.VMEM((E * LANES,), jnp.float32),
            pltpu.VMEM((E, CB, D), jnp.float32),
            pltpu.VMEM((E, CB, D), jnp.float32),
            pltpu.VMEM((N, CB, D), jnp.float32),
            pltpu.VMEM((N, CB, D), jnp.float32),
            pltpu.SemaphoreType.DMA,
            pltpu.SemaphoreType.DMA,
            pltpu.SemaphoreType.DMA,
            pltpu.SemaphoreType.DMA,
        ],
        compiler_params=pltpu.CompilerParams(use_tc_tiling_on_sc=True),
    )
    return f(x_t, w_bcast)


def _tc_fused_body(x_ref, w_ref, wt_ref, b_ref, o_ref):
    for n in range(N):
        acc = None
        for e in _EDGES_OF_NODE[n]:
            t = x_ref[e] * w_ref[0, e]
            acc = t if acc is None else acc + t
        r = jnp.dot(acc, wt_ref[:], preferred_element_type=jnp.float32)
        o_ref[n] = jnp.maximum(r + b_ref[:], 0.0)


def _tc_fused(x_t, w_sp2d, wt, b2d):
    nb = BTC // BMB
    return pl.pallas_call(
        _tc_fused_body,
        grid=(nb,),
        in_specs=[
            pl.BlockSpec((E, BMB, D), lambda j: (0, BSC // BMB + j, 0)),
            pl.BlockSpec((1, E), lambda j: (0, 0)),
            pl.BlockSpec((D, D), lambda j: (0, 0)),
            pl.BlockSpec((1, D), lambda j: (0, 0)),
        ],
        out_specs=pl.BlockSpec((N, BMB, D), lambda j: (0, BSC // BMB + j, 0)),
        out_shape=jax.ShapeDtypeStruct((N, B, D), jnp.float32),
        compiler_params=pltpu.CompilerParams(
            dimension_semantics=("arbitrary",)),
    )(x_t, w_sp2d, wt, b2d)


def _mm_chunk_body(prev_ref, x_ref, wt_ref, b_ref, o_ref):
    del prev_ref
    acc = jnp.dot(x_ref[0], wt_ref[:], preferred_element_type=jnp.float32)
    o_ref[0] = jnp.maximum(acc + b_ref[:], 0.0)


def _tc_chunk(p, nf, wt, b2d, prev):
    jb = BC // BMC
    return pl.pallas_call(
        _mm_chunk_body,
        grid=(N, jb),
        in_specs=[
            pl.BlockSpec(memory_space=pl.ANY),
            pl.BlockSpec((1, BMC, D), lambda n, j: (n, j, 0)),
            pl.BlockSpec((D, D), lambda n, j: (0, 0)),
            pl.BlockSpec((1, D), lambda n, j: (0, 0)),
        ],
        out_specs=pl.BlockSpec((1, BMC, D),
                               lambda n, j: (n, (p * BC) // BMC + j, 0)),
        out_shape=jax.ShapeDtypeStruct((N, B, D), jnp.float32),
        input_output_aliases={0: 0},
        compiler_params=pltpu.CompilerParams(
            dimension_semantics=("arbitrary", "arbitrary")),
    )(prev, nf, wt, b2d)


def kernel(edge_feats, edge_weights, W, b):
    w_sp = jax.nn.softplus(edge_weights.astype(jnp.float32))
    w_bcast = jnp.broadcast_to(w_sp[:, None], (E, LANES)).reshape(-1)
    w_sp2d = w_sp.reshape(1, E)
    x_t = jnp.transpose(edge_feats, (1, 0, 2))
    wt = W.T
    b2d = b.reshape(1, D)
    nfs = [_sc_aggregate(p, x_t, w_bcast) for p in range(P)]
    out3d = _tc_fused(x_t, w_sp2d, wt, b2d)
    for p in range(P):
        out3d = _tc_chunk(p, nfs[p], wt, b2d, out3d)
    return jnp.transpose(out3d, (1, 0, 2))
